# baseline (speedup 1.0000x reference)
import math

import jax
import jax.numpy as jnp
from jax import lax
from jax.experimental import pallas as pl
from jax.experimental.pallas import tpu as pltpu
from jax.experimental.pallas import tpu_sc as plsc

N_USERS = 50000
N_ITEMS = 50000
D = 128
E = 1600000

NC, NS = 2, 16
NW = NC * NS
EW = E // NW
B1 = 200
SUB = 40
NSUB = B1 // SUB

CHUNK = 32
NCHUNK = D // CHUNK
B2 = 256
NBLK = E // B2
ACC_ROWS = 51200
TROWS = ACC_ROWS // NS
ZROWS = 64

_INV_SQRT_D = 1.0 / math.sqrt(float(D))


_TBLK = 400


def _transform_body(x_ref, wu_ref, bu_ref, wd_ref, bd_ref, o_ref):
    i = pl.program_id(0)
    x = x_ref[...]
    yu = jnp.maximum(jnp.dot(x, wu_ref[...], preferred_element_type=jnp.float32)
                     + bu_ref[...], 0.0)
    yd = jnp.maximum(jnp.dot(x, wd_ref[...], preferred_element_type=jnp.float32)
                     + bd_ref[...], 0.0)
    o_ref[...] = jnp.where(i < (N_USERS // _TBLK), yu, yd)


def _transform(feat, wu_t, bu, wd_t, bd):
    n = feat.shape[0]
    return pl.pallas_call(
        _transform_body,
        grid=(n // _TBLK,),
        in_specs=[
            pl.BlockSpec((_TBLK, D), lambda i: (i, 0)),
            pl.BlockSpec((D, D), lambda i: (0, 0)),
            pl.BlockSpec((1, D), lambda i: (0, 0)),
            pl.BlockSpec((D, D), lambda i: (0, 0)),
            pl.BlockSpec((1, D), lambda i: (0, 0)),
        ],
        out_specs=pl.BlockSpec((_TBLK, D), lambda i: (i, 0)),
        out_shape=jax.ShapeDtypeStruct((n, D), jnp.float32),
    )(feat, wu_t, bu, wd_t, bd)


def _softmax_body(l_ref, p_ref):
    l = l_ref[...] * _INV_SQRT_D
    m = jnp.max(l)
    ex = jnp.exp(l - m)
    z = jnp.sum(ex)
    p_ref[...] = ex / z


def _softmax(logits2d):
    return pl.pallas_call(
        _softmax_body,
        out_shape=jax.ShapeDtypeStruct(logits2d.shape, jnp.float32),
    )(logits2d)


def _logits_body(srcg3, dstg3, feat_hbm, out_hbm,
                 idxu_v, idxv_v, urows_v, vrows_v, lbuf_v, semi, semg, semo):
    wid = lax.axis_index("s") * NC + lax.axis_index("c")
    row0 = wid * (EW // B1)
    nr = EW // B1

    def issue_idx(r, slot):
        pltpu.async_copy(srcg3.at[pl.ds(row0 + r, 1)],
                         idxu_v.at[pl.ds(slot, 1)], semi)
        pltpu.async_copy(dstg3.at[pl.ds(row0 + r, 1)],
                         idxv_v.at[pl.ds(slot, 1)], semi)

    def wait_idx():
        pltpu.make_async_copy(srcg3.at[pl.ds(row0, 1)],
                              idxu_v.at[pl.ds(0, 1)], semi).wait()
        pltpu.make_async_copy(dstg3.at[pl.ds(row0, 1)],
                              idxv_v.at[pl.ds(0, 1)], semi).wait()

    def issue_gathers(slot):
        for j in range(NSUB):
            pltpu.async_copy(
                feat_hbm.at[idxu_v.at[slot, 0, pl.ds(j * SUB, SUB)]],
                urows_v.at[slot, pl.ds(j * SUB, SUB)], semg)
            pltpu.async_copy(
                feat_hbm.at[idxv_v.at[slot, 0, pl.ds(j * SUB, SUB)]],
                vrows_v.at[slot, pl.ds(j * SUB, SUB)], semg)

    def wait_gathers():
        for j in range(2 * NSUB):
            pltpu.make_async_copy(
                feat_hbm.at[idxu_v.at[0, 0, pl.ds(0, SUB)]],
                urows_v.at[0, pl.ds(0, SUB)], semg).wait()

    def wait_out():
        pltpu.make_async_copy(lbuf_v.at[0], out_hbm.at[pl.ds(0, B1)],
                              semo).wait()

    issue_idx(0, 0)
    issue_idx(1, 1)
    wait_idx()
    issue_gathers(0)

    def round_body(r, _):
        slot = lax.rem(r, 2)
        wait_gathers()

        def edge_body(e, _):
            prods = [urows_v[slot, e, pl.ds(16 * si, 16)]
                     * vrows_v[slot, e, pl.ds(16 * si, 16)] for si in range(8)]
            while len(prods) > 1:
                prods = [prods[i] + prods[i + 1]
                         for i in range(0, len(prods), 2)]
            lbuf_v[slot, e, pl.ds(0, 16)] = prods[0]
            return 0

        lax.fori_loop(0, B1, edge_body, 0, unroll=4)
        pltpu.async_copy(lbuf_v.at[slot],
                         out_hbm.at[pl.ds((row0 + r) * B1, B1)], semo)

        @pl.when(r >= 1)
        def _():
            wait_out()

        @pl.when(r + 1 < nr)
        def _():
            wait_idx()
            issue_gathers(1 - slot)

        @pl.when(r + 2 < nr)
        def _():
            issue_idx(r + 2, slot)

        return 0

    lax.fori_loop(0, nr, round_body, 0, unroll=False)
    wait_out()


def _edge_logits(srcg3, dstg3, feat):
    mesh = plsc.VectorSubcoreMesh(core_axis_name="c", subcore_axis_name="s")
    return pl.kernel(
        _logits_body,
        out_type=jax.ShapeDtypeStruct((E, 16), jnp.float32),
        mesh=mesh,
        compiler_params=pltpu.CompilerParams(use_tc_tiling_on_sc=False),
        scratch_types=[
            pltpu.VMEM((2, 1, B1), jnp.int32),
            pltpu.VMEM((2, 1, B1), jnp.int32),
            pltpu.VMEM((2, B1, D), jnp.float32),
            pltpu.VMEM((2, B1, D), jnp.float32),
            pltpu.VMEM((2, B1, 16), jnp.float32),
            pltpu.SemaphoreType.DMA,
            pltpu.SemaphoreType.DMA,
            pltpu.SemaphoreType.DMA,
        ],
    )(srcg3, dstg3, feat)


_RCOLS = 1024
_RBLK = 200


def _rowsum_body(x_ref, s_ref, o_ref):
    o_ref[...] = jnp.dot(x_ref[...], s_ref[...],
                         preferred_element_type=jnp.float32)


def _rowsum(part2d, sel):
    n = part2d.shape[0]
    return pl.pallas_call(
        _rowsum_body,
        grid=(n // _RBLK,),
        in_specs=[
            pl.BlockSpec((_RBLK, _RCOLS), lambda i: (i, 0)),
            pl.BlockSpec((_RCOLS, _RCOLS // 16), lambda i: (0, 0)),
        ],
        out_specs=pl.BlockSpec((_RBLK, _RCOLS // 16), lambda i: (i, 0)),
        out_shape=jax.ShapeDtypeStruct((n, _RCOLS // 16), jnp.float32),
    )(part2d, sel)


def _accum_body(src1d, dstl1d, p1d, src3d, dstl3d,
                fsrc_list, fdst_list, out4_hbm,
                gidx_v, sidx_v, grows_v, p_v, zbuf_v, acc_sh,
                semi, semg, semsc):
    core = lax.axis_index("c")
    s = lax.axis_index("s")
    lane = lax.iota(jnp.int32, 16)
    zeros16 = jnp.zeros((16,), jnp.float32)

    def zrow(i, _):
        zbuf_v[i, pl.ds(0, 16)] = zeros16
        zbuf_v[i, pl.ds(16, 16)] = zeros16
        return 0

    lax.fori_loop(0, ZROWS, zrow, 0, unroll=False)

    nb = (NBLK - s + NS - 1) // NS
    _NJ = B2 // 128

    def edge_sweep(gather_ref, gidx_hbm, sidx_hbm):
        def issue_idx(t, gslot, sslot):
            b = s + t * NS
            pltpu.async_copy(gidx_hbm.at[pl.ds(b, 1)],
                             gidx_v.at[pl.ds(gslot, 1)], semi)
            pltpu.async_copy(p1d.at[pl.ds(b, 1)], p_v.at[pl.ds(gslot, 1)], semi)
            pltpu.async_copy(sidx_hbm.at[pl.ds(b * _NJ, _NJ)],
                             sidx_v.at[pl.ds(sslot * _NJ, _NJ)], semi)

        def wait_idx():
            pltpu.make_async_copy(gidx_hbm.at[pl.ds(s, 1)],
                                  gidx_v.at[pl.ds(0, 1)], semi).wait()
            pltpu.make_async_copy(p1d.at[pl.ds(s, 1)],
                                  p_v.at[pl.ds(0, 1)], semi).wait()
            pltpu.make_async_copy(sidx_hbm.at[pl.ds(s, _NJ)],
                                  sidx_v.at[pl.ds(0, _NJ)], semi).wait()

        def issue_gather(slot):
            for j in range(_NJ):
                pltpu.async_copy(
                    gather_ref.at[gidx_v.at[slot, 0, pl.ds(j * 128, 128)]],
                    grows_v.at[slot, pl.ds(j * 128, 128)], semg)

        def wait_gather():
            for j in range(_NJ):
                pltpu.make_async_copy(
                    gather_ref.at[gidx_v.at[0, 0, pl.ds(j * 128, 128)]],
                    grows_v.at[0, pl.ds(j * 128, 128)], semg).wait()

        def issue_scatter(slot, s3):
            for j in range(_NJ):
                pltpu.async_copy(grows_v.at[slot, pl.ds(j * 128, 128)],
                                 acc_sh.at[sidx_v.at[s3 * _NJ + j, 0]], semsc,
                                 add=True)

        def wait_scatter():
            for j in range(_NJ):
                pltpu.make_async_copy(grows_v.at[0, pl.ds(j * 128, 128)],
                                      acc_sh.at[sidx_v.at[j, 0]], semsc).wait()

        def scale(slot):
            def scale_group(g, _):
                pv16 = p_v[slot, 0, pl.ds(g * 16, 16)]
                base = g * 16
                for k in range(16):
                    pv = pv16[k]
                    for si in range(CHUNK // 16):
                        sl = pl.ds(16 * si, 16)
                        grows_v[slot, base + k, sl] = (
                            grows_v[slot, base + k, sl] * pv)
                return 0

            lax.fori_loop(0, B2 // 16, scale_group, 0, unroll=2)

        issue_idx(0, 0, 0)
        issue_idx(1, 1, 1)
        wait_idx()
        issue_gather(0)

        def blk(t, _):
            slot = lax.rem(t, 2)
            s3 = lax.rem(t, 3)
            wait_gather()
            scale(slot)
            issue_scatter(slot, s3)

            @pl.when(t >= 1)
            def _():
                wait_scatter()

            @pl.when(t + 1 < nb)
            def _():
                wait_idx()
                issue_gather(1 - slot)

            @pl.when(t + 2 < nb)
            def _():
                issue_idx(t + 2, slot, lax.rem(t + 2, 3))

            return 0

        lax.fori_loop(0, nb, blk, 0, unroll=False)
        wait_scatter()

    for c in range(NCHUNK):
        for k in range(TROWS // ZROWS):
            pltpu.sync_copy(zbuf_v, acc_sh.at[pl.ds(s * TROWS + k * ZROWS, ZROWS)])
        plsc.subcore_barrier()

        @pl.when(core == 0)
        def _():
            edge_sweep(fsrc_list[c], src1d, dstl3d)

        @pl.when(core == 1)
        def _():
            edge_sweep(fdst_list[c], dstl1d, src3d)

        plsc.subcore_barrier()

        rowbase = jnp.where(core == 0, N_USERS, 0)

        @pl.when(s < NS - 1)
        def _():
            pltpu.sync_copy(acc_sh.at[pl.ds(s * TROWS, TROWS)],
                            out4_hbm.at[c, pl.ds(rowbase + s * TROWS, TROWS)])

        @pl.when(s == NS - 1)
        def _():
            last = N_USERS - (NS - 1) * TROWS
            pltpu.sync_copy(acc_sh.at[pl.ds((NS - 1) * TROWS, last)],
                            out4_hbm.at[c, pl.ds(rowbase + (NS - 1) * TROWS, last)])

        plsc.subcore_barrier()


def _aggregate(src1d, dstl1d, p1d, src3d, dstl3d, fsrc_list, fdst_list):
    mesh = plsc.VectorSubcoreMesh(core_axis_name="c", subcore_axis_name="s")

    def body(src1d_r, dstl1d_r, p1d_r, src3d_r, dstl3d_r,
             f0, f1, f2, f3, g0, g1, g2, g3, out4_r,
             gidx_v, sidx_v, grows_v, p_v, zbuf_v, acc_sh, semi, semg, semsc):
        _accum_body(src1d_r, dstl1d_r, p1d_r, src3d_r, dstl3d_r,
                    [f0, f1, f2, f3], [g0, g1, g2, g3], out4_r,
                    gidx_v, sidx_v, grows_v, p_v, zbuf_v, acc_sh,
                    semi, semg, semsc)

    return pl.kernel(
        body,
        out_type=jax.ShapeDtypeStruct((NCHUNK, 2 * N_USERS, CHUNK), jnp.float32),
        mesh=mesh,
        compiler_params=pltpu.CompilerParams(use_tc_tiling_on_sc=False),
        scratch_types=[
            pltpu.VMEM((2, 1, B2), jnp.int32),
            pltpu.VMEM((3 * (B2 // 128), 1, 128), jnp.int32),
            pltpu.VMEM((2, B2, CHUNK), jnp.float32),
            pltpu.VMEM((2, 1, B2), jnp.float32),
            pltpu.VMEM((ZROWS, CHUNK), jnp.float32),
            pltpu.VMEM_SHARED((ACC_ROWS, CHUNK), jnp.float32),
            pltpu.SemaphoreType.DMA,
            pltpu.SemaphoreType.DMA,
            pltpu.SemaphoreType.DMA,
        ],
    )(src1d, dstl1d, p1d, src3d, dstl3d, *fsrc_list, *fdst_list)


def kernel(feat, edge_index, user_ids, item_ids, W_src, b_src, W_dst, b_dst):
    feat = feat.astype(jnp.float32)
    src = edge_index[0].astype(jnp.int32)
    dstl = edge_index[1].astype(jnp.int32)
    dstg = dstl + N_USERS

    t_all = _transform(feat, W_src.T, b_src.reshape(1, D), W_dst.T,
                       b_dst.reshape(1, D))

    lpart = _edge_logits(src.reshape(E // B1, 1, B1),
                         dstg.reshape(E // B1, 1, B1), feat)
    sel = jnp.repeat(jnp.eye(_RCOLS // 16, dtype=jnp.float32), 16, axis=0)
    logits2d = _rowsum(lpart.reshape(E * 16 // _RCOLS, _RCOLS), sel)
    p = _softmax(logits2d).reshape(-1)

    fsrc_list = [t_all[:N_USERS, c * CHUNK:(c + 1) * CHUNK] for c in range(NCHUNK)]
    fdst_list = [t_all[N_USERS:, c * CHUNK:(c + 1) * CHUNK] for c in range(NCHUNK)]
    src_g3 = src.reshape(NBLK, 1, B2)
    dstl_g3 = dstl.reshape(NBLK, 1, B2)
    p3 = p.reshape(NBLK, 1, B2)
    src_s3 = src.reshape(NBLK * (B2 // 128), 1, 128)
    dstl_s3 = dstl.reshape(NBLK * (B2 // 128), 1, 128)
    out4 = _aggregate(src_g3, dstl_g3, p3, src_s3, dstl_s3,
                      fsrc_list, fdst_list)
    return out4.transpose(1, 0, 2).reshape(2 * N_USERS, D)


# speedup vs baseline: 1.2304x; 1.2304x over previous
import math

import jax
import jax.numpy as jnp
from jax import lax
from jax.experimental import pallas as pl
from jax.experimental.pallas import tpu as pltpu
from jax.experimental.pallas import tpu_sc as plsc

N_USERS = 50000
N_ITEMS = 50000
D = 128
E = 1600000

NC, NS = 2, 16
NW = NC * NS
EW = E // NW
B1 = 200
SUB = 40
NSUB = B1 // SUB

CHUNK = 32
NCHUNK = D // CHUNK
B2 = 256
NBLK = E // B2
ACC_ROWS = 51200
TROWS = ACC_ROWS // NS
ZROWS = 64

_INV_SQRT_D = 1.0 / math.sqrt(float(D))


_TBLK = 400


def _transform_body(x_ref, wu_ref, bu_ref, wd_ref, bd_ref, o_ref):
    i = pl.program_id(0)
    x = x_ref[...]
    yu = jnp.maximum(jnp.dot(x, wu_ref[...], preferred_element_type=jnp.float32)
                     + bu_ref[...], 0.0)
    yd = jnp.maximum(jnp.dot(x, wd_ref[...], preferred_element_type=jnp.float32)
                     + bd_ref[...], 0.0)
    o_ref[...] = jnp.where(i < (N_USERS // _TBLK), yu, yd)


def _transform(feat, wu_t, bu, wd_t, bd):
    n = feat.shape[0]
    return pl.pallas_call(
        _transform_body,
        grid=(n // _TBLK,),
        in_specs=[
            pl.BlockSpec((_TBLK, D), lambda i: (i, 0)),
            pl.BlockSpec((D, D), lambda i: (0, 0)),
            pl.BlockSpec((1, D), lambda i: (0, 0)),
            pl.BlockSpec((D, D), lambda i: (0, 0)),
            pl.BlockSpec((1, D), lambda i: (0, 0)),
        ],
        out_specs=pl.BlockSpec((_TBLK, D), lambda i: (i, 0)),
        out_shape=jax.ShapeDtypeStruct((n, D), jnp.float32),
    )(feat, wu_t, bu, wd_t, bd)


def _softmax_body(l_ref, p_ref):
    l = l_ref[...] * _INV_SQRT_D
    m = jnp.max(l)
    ex = jnp.exp(l - m)
    z = jnp.sum(ex)
    p_ref[...] = ex / z


def _softmax(logits2d):
    return pl.pallas_call(
        _softmax_body,
        out_shape=jax.ShapeDtypeStruct(logits2d.shape, jnp.float32),
    )(logits2d)


def _logits_body(srcg3, dstg3, feat_hbm, out_hbm,
                 idxu_v, idxv_v, urows_v, vrows_v, lbuf_v, semi, semg, semo):
    wid = lax.axis_index("s") * NC + lax.axis_index("c")
    row0 = wid * (EW // B1)
    nr = EW // B1

    def issue_idx(r, slot):
        pltpu.async_copy(srcg3.at[pl.ds(row0 + r, 1)],
                         idxu_v.at[pl.ds(slot, 1)], semi)
        pltpu.async_copy(dstg3.at[pl.ds(row0 + r, 1)],
                         idxv_v.at[pl.ds(slot, 1)], semi)

    def wait_idx():
        pltpu.make_async_copy(srcg3.at[pl.ds(row0, 1)],
                              idxu_v.at[pl.ds(0, 1)], semi).wait()
        pltpu.make_async_copy(dstg3.at[pl.ds(row0, 1)],
                              idxv_v.at[pl.ds(0, 1)], semi).wait()

    def issue_gathers(slot):
        for j in range(NSUB):
            pltpu.async_copy(
                feat_hbm.at[idxu_v.at[slot, 0, pl.ds(j * SUB, SUB)]],
                urows_v.at[slot, pl.ds(j * SUB, SUB)], semg)
            pltpu.async_copy(
                feat_hbm.at[idxv_v.at[slot, 0, pl.ds(j * SUB, SUB)]],
                vrows_v.at[slot, pl.ds(j * SUB, SUB)], semg)

    def wait_gathers():
        for j in range(2 * NSUB):
            pltpu.make_async_copy(
                feat_hbm.at[idxu_v.at[0, 0, pl.ds(0, SUB)]],
                urows_v.at[0, pl.ds(0, SUB)], semg).wait()

    def wait_out():
        pltpu.make_async_copy(lbuf_v.at[0], out_hbm.at[pl.ds(0, B1)],
                              semo).wait()

    issue_idx(0, 0)
    issue_idx(1, 1)
    wait_idx()
    issue_gathers(0)

    def round_body(r, _):
        slot = lax.rem(r, 2)
        wait_gathers()

        def edge_body(e, _):
            prods = [urows_v[slot, e, pl.ds(16 * si, 16)]
                     * vrows_v[slot, e, pl.ds(16 * si, 16)] for si in range(8)]
            while len(prods) > 1:
                prods = [prods[i] + prods[i + 1]
                         for i in range(0, len(prods), 2)]
            lbuf_v[slot, e, pl.ds(0, 16)] = prods[0]
            return 0

        lax.fori_loop(0, B1, edge_body, 0, unroll=False)
        pltpu.async_copy(lbuf_v.at[slot],
                         out_hbm.at[pl.ds((row0 + r) * B1, B1)], semo)

        @pl.when(r >= 1)
        def _():
            wait_out()

        @pl.when(r + 1 < nr)
        def _():
            wait_idx()
            issue_gathers(1 - slot)

        @pl.when(r + 2 < nr)
        def _():
            issue_idx(r + 2, slot)

        return 0

    lax.fori_loop(0, nr, round_body, 0, unroll=False)
    wait_out()


def _edge_logits(srcg3, dstg3, feat):
    mesh = plsc.VectorSubcoreMesh(core_axis_name="c", subcore_axis_name="s")
    return pl.kernel(
        _logits_body,
        out_type=jax.ShapeDtypeStruct((E, 16), jnp.float32),
        mesh=mesh,
        compiler_params=pltpu.CompilerParams(use_tc_tiling_on_sc=False),
        scratch_types=[
            pltpu.VMEM((2, 1, B1), jnp.int32),
            pltpu.VMEM((2, 1, B1), jnp.int32),
            pltpu.VMEM((2, B1, D), jnp.float32),
            pltpu.VMEM((2, B1, D), jnp.float32),
            pltpu.VMEM((2, B1, 16), jnp.float32),
            pltpu.SemaphoreType.DMA,
            pltpu.SemaphoreType.DMA,
            pltpu.SemaphoreType.DMA,
        ],
    )(srcg3, dstg3, feat)


_RCOLS = 1024
_RBLK = 200


def _rowsum_body(x_ref, s_ref, o_ref):
    o_ref[...] = jnp.dot(x_ref[...], s_ref[...],
                         preferred_element_type=jnp.float32)


def _rowsum(part2d, sel):
    n = part2d.shape[0]
    return pl.pallas_call(
        _rowsum_body,
        grid=(n // _RBLK,),
        in_specs=[
            pl.BlockSpec((_RBLK, _RCOLS), lambda i: (i, 0)),
            pl.BlockSpec((_RCOLS, _RCOLS // 16), lambda i: (0, 0)),
        ],
        out_specs=pl.BlockSpec((_RBLK, _RCOLS // 16), lambda i: (i, 0)),
        out_shape=jax.ShapeDtypeStruct((n, _RCOLS // 16), jnp.float32),
    )(part2d, sel)


def _accum_body(src1d, dstl1d, p1d, src3d, dstl3d,
                fsrc_list, fdst_list, out4_hbm,
                gidx_v, sidx_v, grows_v, p_v, zbuf_v, acc_sh,
                semi, semg, semsc):
    core = lax.axis_index("c")
    s = lax.axis_index("s")
    lane = lax.iota(jnp.int32, 16)
    zeros16 = jnp.zeros((16,), jnp.float32)

    def zrow(i, _):
        zbuf_v[i, pl.ds(0, 16)] = zeros16
        zbuf_v[i, pl.ds(16, 16)] = zeros16
        return 0

    lax.fori_loop(0, ZROWS, zrow, 0, unroll=False)

    nb = (NBLK - s + NS - 1) // NS
    _NJ = B2 // 128

    def edge_sweep(gather_ref, gidx_hbm, sidx_hbm):
        def issue_idx(t, gslot, sslot):
            b = s + t * NS
            pltpu.async_copy(gidx_hbm.at[pl.ds(b, 1)],
                             gidx_v.at[pl.ds(gslot, 1)], semi)
            pltpu.async_copy(p1d.at[pl.ds(b, 1)], p_v.at[pl.ds(gslot, 1)], semi)
            pltpu.async_copy(sidx_hbm.at[pl.ds(b * _NJ, _NJ)],
                             sidx_v.at[pl.ds(sslot * _NJ, _NJ)], semi)

        def wait_idx():
            pltpu.make_async_copy(gidx_hbm.at[pl.ds(s, 1)],
                                  gidx_v.at[pl.ds(0, 1)], semi).wait()
            pltpu.make_async_copy(p1d.at[pl.ds(s, 1)],
                                  p_v.at[pl.ds(0, 1)], semi).wait()
            pltpu.make_async_copy(sidx_hbm.at[pl.ds(s, _NJ)],
                                  sidx_v.at[pl.ds(0, _NJ)], semi).wait()

        def issue_gather(slot):
            for j in range(_NJ):
                pltpu.async_copy(
                    gather_ref.at[gidx_v.at[slot, 0, pl.ds(j * 128, 128)]],
                    grows_v.at[slot, pl.ds(j * 128, 128)], semg)

        def wait_gather():
            for j in range(_NJ):
                pltpu.make_async_copy(
                    gather_ref.at[gidx_v.at[0, 0, pl.ds(j * 128, 128)]],
                    grows_v.at[0, pl.ds(j * 128, 128)], semg).wait()

        def issue_scatter(slot, s3):
            for j in range(_NJ):
                pltpu.async_copy(grows_v.at[slot, pl.ds(j * 128, 128)],
                                 acc_sh.at[sidx_v.at[s3 * _NJ + j, 0]], semsc,
                                 add=True)

        def wait_scatter():
            for j in range(_NJ):
                pltpu.make_async_copy(grows_v.at[0, pl.ds(j * 128, 128)],
                                      acc_sh.at[sidx_v.at[j, 0]], semsc).wait()

        def scale(slot):
            def scale_group(g, _):
                pv16 = p_v[slot, 0, pl.ds(g * 16, 16)]
                base = g * 16
                for k in range(16):
                    pv = pv16[k]
                    for si in range(CHUNK // 16):
                        sl = pl.ds(16 * si, 16)
                        grows_v[slot, base + k, sl] = (
                            grows_v[slot, base + k, sl] * pv)
                return 0

            lax.fori_loop(0, B2 // 16, scale_group, 0, unroll=False)

        issue_idx(0, 0, 0)
        issue_idx(1, 1, 1)
        wait_idx()
        issue_gather(0)

        def blk(t, _):
            slot = lax.rem(t, 2)
            s3 = lax.rem(t, 3)
            wait_gather()
            scale(slot)
            issue_scatter(slot, s3)

            @pl.when(t >= 1)
            def _():
                wait_scatter()

            @pl.when(t + 1 < nb)
            def _():
                wait_idx()
                issue_gather(1 - slot)

            @pl.when(t + 2 < nb)
            def _():
                issue_idx(t + 2, slot, lax.rem(t + 2, 3))

            return 0

        lax.fori_loop(0, nb, blk, 0, unroll=False)
        wait_scatter()

    for c in range(NCHUNK):
        for k in range(TROWS // ZROWS):
            pltpu.sync_copy(zbuf_v, acc_sh.at[pl.ds(s * TROWS + k * ZROWS, ZROWS)])
        plsc.subcore_barrier()

        @pl.when(core == 0)
        def _():
            edge_sweep(fsrc_list[c], src1d, dstl3d)

        @pl.when(core == 1)
        def _():
            edge_sweep(fdst_list[c], dstl1d, src3d)

        plsc.subcore_barrier()

        rowbase = jnp.where(core == 0, N_USERS, 0)

        @pl.when(s < NS - 1)
        def _():
            pltpu.sync_copy(acc_sh.at[pl.ds(s * TROWS, TROWS)],
                            out4_hbm.at[c, pl.ds(rowbase + s * TROWS, TROWS)])

        @pl.when(s == NS - 1)
        def _():
            last = N_USERS - (NS - 1) * TROWS
            pltpu.sync_copy(acc_sh.at[pl.ds((NS - 1) * TROWS, last)],
                            out4_hbm.at[c, pl.ds(rowbase + (NS - 1) * TROWS, last)])

        plsc.subcore_barrier()


def _aggregate(src1d, dstl1d, p1d, src3d, dstl3d, fsrc_list, fdst_list):
    mesh = plsc.VectorSubcoreMesh(core_axis_name="c", subcore_axis_name="s")

    def body(src1d_r, dstl1d_r, p1d_r, src3d_r, dstl3d_r,
             f0, f1, f2, f3, g0, g1, g2, g3, out4_r,
             gidx_v, sidx_v, grows_v, p_v, zbuf_v, acc_sh, semi, semg, semsc):
        _accum_body(src1d_r, dstl1d_r, p1d_r, src3d_r, dstl3d_r,
                    [f0, f1, f2, f3], [g0, g1, g2, g3], out4_r,
                    gidx_v, sidx_v, grows_v, p_v, zbuf_v, acc_sh,
                    semi, semg, semsc)

    return pl.kernel(
        body,
        out_type=jax.ShapeDtypeStruct((NCHUNK, 2 * N_USERS, CHUNK), jnp.float32),
        mesh=mesh,
        compiler_params=pltpu.CompilerParams(use_tc_tiling_on_sc=False),
        scratch_types=[
            pltpu.VMEM((2, 1, B2), jnp.int32),
            pltpu.VMEM((3 * (B2 // 128), 1, 128), jnp.int32),
            pltpu.VMEM((2, B2, CHUNK), jnp.float32),
            pltpu.VMEM((2, 1, B2), jnp.float32),
            pltpu.VMEM((ZROWS, CHUNK), jnp.float32),
            pltpu.VMEM_SHARED((ACC_ROWS, CHUNK), jnp.float32),
            pltpu.SemaphoreType.DMA,
            pltpu.SemaphoreType.DMA,
            pltpu.SemaphoreType.DMA,
        ],
    )(src1d, dstl1d, p1d, src3d, dstl3d, *fsrc_list, *fdst_list)


def kernel(feat, edge_index, user_ids, item_ids, W_src, b_src, W_dst, b_dst):
    feat = feat.astype(jnp.float32)
    src = edge_index[0].astype(jnp.int32)
    dstl = edge_index[1].astype(jnp.int32)
    dstg = dstl + N_USERS

    t_all = _transform(feat, W_src.T, b_src.reshape(1, D), W_dst.T,
                       b_dst.reshape(1, D))

    lpart = _edge_logits(src.reshape(E // B1, 1, B1),
                         dstg.reshape(E // B1, 1, B1), feat)
    sel = jnp.repeat(jnp.eye(_RCOLS // 16, dtype=jnp.float32), 16, axis=0)
    logits2d = _rowsum(lpart.reshape(E * 16 // _RCOLS, _RCOLS), sel)
    p = _softmax(logits2d).reshape(-1)

    fsrc_list = [t_all[:N_USERS, c * CHUNK:(c + 1) * CHUNK] for c in range(NCHUNK)]
    fdst_list = [t_all[N_USERS:, c * CHUNK:(c + 1) * CHUNK] for c in range(NCHUNK)]
    src_g3 = src.reshape(NBLK, 1, B2)
    dstl_g3 = dstl.reshape(NBLK, 1, B2)
    p3 = p.reshape(NBLK, 1, B2)
    src_s3 = src.reshape(NBLK * (B2 // 128), 1, 128)
    dstl_s3 = dstl.reshape(NBLK * (B2 // 128), 1, 128)
    out4 = _aggregate(src_g3, dstl_g3, p3, src_s3, dstl_s3,
                      fsrc_list, fdst_list)
    return out4.transpose(1, 0, 2).reshape(2 * N_USERS, D)


# speedup vs baseline: 1.6549x; 1.3450x over previous
import math

import jax
import jax.numpy as jnp
from jax import lax
from jax.experimental import pallas as pl
from jax.experimental.pallas import tpu as pltpu
from jax.experimental.pallas import tpu_sc as plsc

N_USERS = 50000
N_ITEMS = 50000
D = 128
E = 1600000

NC, NS = 2, 16
NW = NC * NS
EW = E // NW
B1 = 200
SUB = 40
NSUB = B1 // SUB

CHUNK = 32
NCHUNK = D // CHUNK
B2 = 256
NBLK = E // B2
ACC_ROWS = 51200
TROWS = ACC_ROWS // NS
ZROWS = 64

_INV_SQRT_D = 1.0 / math.sqrt(float(D))


_TBLK = 400


def _transform_body(x_ref, wu_ref, bu_ref, wd_ref, bd_ref, o_ref):
    i = pl.program_id(0)
    x = x_ref[...]
    yu = jnp.maximum(jnp.dot(x, wu_ref[...], preferred_element_type=jnp.float32)
                     + bu_ref[...], 0.0)
    yd = jnp.maximum(jnp.dot(x, wd_ref[...], preferred_element_type=jnp.float32)
                     + bd_ref[...], 0.0)
    o_ref[...] = jnp.where(i < (N_USERS // _TBLK), yu, yd)


def _transform(feat, wu_t, bu, wd_t, bd):
    n = feat.shape[0]
    return pl.pallas_call(
        _transform_body,
        grid=(n // _TBLK,),
        in_specs=[
            pl.BlockSpec((_TBLK, D), lambda i: (i, 0)),
            pl.BlockSpec((D, D), lambda i: (0, 0)),
            pl.BlockSpec((1, D), lambda i: (0, 0)),
            pl.BlockSpec((D, D), lambda i: (0, 0)),
            pl.BlockSpec((1, D), lambda i: (0, 0)),
        ],
        out_specs=pl.BlockSpec((_TBLK, D), lambda i: (i, 0)),
        out_shape=jax.ShapeDtypeStruct((n, D), jnp.float32),
    )(feat, wu_t, bu, wd_t, bd)


def _softmax_body(l_ref, p_ref):
    l = l_ref[...] * _INV_SQRT_D
    m = jnp.max(l)
    ex = jnp.exp(l - m)
    z = jnp.sum(ex)
    p_ref[...] = ex / z


def _softmax(logits2d):
    return pl.pallas_call(
        _softmax_body,
        out_shape=jax.ShapeDtypeStruct(logits2d.shape, jnp.float32),
    )(logits2d)


def _logits_body(srcg3, dstg3, feat_hbm, out_hbm,
                 idxu_v, idxv_v, urows_v, vrows_v, lbuf_v, semi, semg, semo):
    wid = lax.axis_index("s") * NC + lax.axis_index("c")
    row0 = wid * (EW // B1)
    nr = EW // B1

    def issue_idx(r, slot):
        pltpu.async_copy(srcg3.at[pl.ds(row0 + r, 1)],
                         idxu_v.at[pl.ds(slot, 1)], semi)
        pltpu.async_copy(dstg3.at[pl.ds(row0 + r, 1)],
                         idxv_v.at[pl.ds(slot, 1)], semi)

    def wait_idx():
        pltpu.make_async_copy(srcg3.at[pl.ds(row0, 1)],
                              idxu_v.at[pl.ds(0, 1)], semi).wait()
        pltpu.make_async_copy(dstg3.at[pl.ds(row0, 1)],
                              idxv_v.at[pl.ds(0, 1)], semi).wait()

    def issue_gathers(slot):
        for j in range(NSUB):
            pltpu.async_copy(
                feat_hbm.at[idxu_v.at[slot, 0, pl.ds(j * SUB, SUB)]],
                urows_v.at[slot, pl.ds(j * SUB, SUB)], semg)
            pltpu.async_copy(
                feat_hbm.at[idxv_v.at[slot, 0, pl.ds(j * SUB, SUB)]],
                vrows_v.at[slot, pl.ds(j * SUB, SUB)], semg)

    def wait_gathers():
        for j in range(2 * NSUB):
            pltpu.make_async_copy(
                feat_hbm.at[idxu_v.at[0, 0, pl.ds(0, SUB)]],
                urows_v.at[0, pl.ds(0, SUB)], semg).wait()

    def wait_out():
        pltpu.make_async_copy(lbuf_v.at[0], out_hbm.at[pl.ds(0, B1)],
                              semo).wait()

    issue_idx(0, 0)
    issue_idx(1, 1)
    wait_idx()
    issue_gathers(0)

    def round_body(r, _):
        slot = lax.rem(r, 2)
        wait_gathers()

        @pl.when(r + 1 < nr)
        def _():
            wait_idx()
            issue_gathers(1 - slot)

        def edge_body(e, _):
            acc = (urows_v[slot, e, pl.ds(0, 16)]
                   * vrows_v[slot, e, pl.ds(0, 16)])
            for si in range(1, 8):
                acc = acc + (urows_v[slot, e, pl.ds(16 * si, 16)]
                             * vrows_v[slot, e, pl.ds(16 * si, 16)])
            lbuf_v[slot, e, pl.ds(0, 16)] = acc
            return 0

        lax.fori_loop(0, B1, edge_body, 0, unroll=False)
        pltpu.async_copy(lbuf_v.at[slot],
                         out_hbm.at[pl.ds((row0 + r) * B1, B1)], semo)

        @pl.when(r >= 1)
        def _():
            wait_out()

        @pl.when(r + 2 < nr)
        def _():
            issue_idx(r + 2, slot)

        return 0

    lax.fori_loop(0, nr, round_body, 0, unroll=False)
    wait_out()


def _edge_logits(srcg3, dstg3, feat):
    mesh = plsc.VectorSubcoreMesh(core_axis_name="c", subcore_axis_name="s")
    return pl.kernel(
        _logits_body,
        out_type=jax.ShapeDtypeStruct((E, 16), jnp.float32),
        mesh=mesh,
        compiler_params=pltpu.CompilerParams(use_tc_tiling_on_sc=False),
        scratch_types=[
            pltpu.VMEM((2, 1, B1), jnp.int32),
            pltpu.VMEM((2, 1, B1), jnp.int32),
            pltpu.VMEM((2, B1, D), jnp.float32),
            pltpu.VMEM((2, B1, D), jnp.float32),
            pltpu.VMEM((2, B1, 16), jnp.float32),
            pltpu.SemaphoreType.DMA,
            pltpu.SemaphoreType.DMA,
            pltpu.SemaphoreType.DMA,
        ],
    )(srcg3, dstg3, feat)


_RCOLS = 1024
_RBLK = 200


def _rowsum_body(x_ref, s_ref, o_ref):
    o_ref[...] = jnp.dot(x_ref[...], s_ref[...],
                         preferred_element_type=jnp.float32)


def _rowsum(part2d, sel):
    n = part2d.shape[0]
    return pl.pallas_call(
        _rowsum_body,
        grid=(n // _RBLK,),
        in_specs=[
            pl.BlockSpec((_RBLK, _RCOLS), lambda i: (i, 0)),
            pl.BlockSpec((_RCOLS, _RCOLS // 16), lambda i: (0, 0)),
        ],
        out_specs=pl.BlockSpec((_RBLK, _RCOLS // 16), lambda i: (i, 0)),
        out_shape=jax.ShapeDtypeStruct((n, _RCOLS // 16), jnp.float32),
    )(part2d, sel)


def _accum_body(src1d, dstl1d, p1d, src3d, dstl3d,
                fsrc_list, fdst_list, out4_hbm,
                gidx_v, sidx_v, grows_v, p_v, zbuf_v, acc_sh,
                semi, semg, semsc):
    core = lax.axis_index("c")
    s = lax.axis_index("s")
    lane = lax.iota(jnp.int32, 16)
    zeros16 = jnp.zeros((16,), jnp.float32)

    def zrow(i, _):
        zbuf_v[i, pl.ds(0, 16)] = zeros16
        zbuf_v[i, pl.ds(16, 16)] = zeros16
        return 0

    lax.fori_loop(0, ZROWS, zrow, 0, unroll=False)

    nb = (NBLK - s + NS - 1) // NS
    _NJ = B2 // 128

    def edge_sweep(gather_ref, gidx_hbm, sidx_hbm):
        def issue_idx(t, gslot, sslot):
            b = s + t * NS
            pltpu.async_copy(gidx_hbm.at[pl.ds(b, 1)],
                             gidx_v.at[pl.ds(gslot, 1)], semi)
            pltpu.async_copy(p1d.at[pl.ds(b, 1)], p_v.at[pl.ds(gslot, 1)], semi)
            pltpu.async_copy(sidx_hbm.at[pl.ds(b * _NJ, _NJ)],
                             sidx_v.at[pl.ds(sslot * _NJ, _NJ)], semi)

        def wait_idx():
            pltpu.make_async_copy(gidx_hbm.at[pl.ds(s, 1)],
                                  gidx_v.at[pl.ds(0, 1)], semi).wait()
            pltpu.make_async_copy(p1d.at[pl.ds(s, 1)],
                                  p_v.at[pl.ds(0, 1)], semi).wait()
            pltpu.make_async_copy(sidx_hbm.at[pl.ds(s, _NJ)],
                                  sidx_v.at[pl.ds(0, _NJ)], semi).wait()

        def issue_gather(slot):
            for j in range(_NJ):
                pltpu.async_copy(
                    gather_ref.at[gidx_v.at[slot, 0, pl.ds(j * 128, 128)]],
                    grows_v.at[slot, pl.ds(j * 128, 128)], semg)

        def wait_gather():
            for j in range(_NJ):
                pltpu.make_async_copy(
                    gather_ref.at[gidx_v.at[0, 0, pl.ds(j * 128, 128)]],
                    grows_v.at[0, pl.ds(j * 128, 128)], semg).wait()

        def issue_scatter(slot, s3):
            for j in range(_NJ):
                pltpu.async_copy(grows_v.at[slot, pl.ds(j * 128, 128)],
                                 acc_sh.at[sidx_v.at[s3 * _NJ + j, 0]], semsc,
                                 add=True)

        def wait_scatter():
            for j in range(_NJ):
                pltpu.make_async_copy(grows_v.at[0, pl.ds(j * 128, 128)],
                                      acc_sh.at[sidx_v.at[j, 0]], semsc).wait()

        def scale(slot):
            def scale_group(g, _):
                pv16 = p_v[slot, 0, pl.ds(g * 16, 16)]
                base = g * 16
                for k in range(16):
                    pv = pv16[k]
                    for si in range(CHUNK // 16):
                        sl = pl.ds(16 * si, 16)
                        grows_v[slot, base + k, sl] = (
                            grows_v[slot, base + k, sl] * pv)
                return 0

            lax.fori_loop(0, B2 // 16, scale_group, 0, unroll=False)

        issue_idx(0, 0, 0)
        issue_idx(1, 1, 1)
        wait_idx()
        issue_gather(0)

        def blk(t, _):
            slot = lax.rem(t, 2)
            s3 = lax.rem(t, 3)
            wait_gather()

            @pl.when(t >= 1)
            def _():
                wait_scatter()

            @pl.when(t + 1 < nb)
            def _():
                wait_idx()
                issue_gather(1 - slot)

            scale(slot)
            issue_scatter(slot, s3)

            @pl.when(t + 2 < nb)
            def _():
                issue_idx(t + 2, slot, lax.rem(t + 2, 3))

            return 0

        lax.fori_loop(0, nb, blk, 0, unroll=False)
        wait_scatter()

    for c in range(NCHUNK):
        for k in range(TROWS // ZROWS):
            pltpu.sync_copy(zbuf_v, acc_sh.at[pl.ds(s * TROWS + k * ZROWS, ZROWS)])
        plsc.subcore_barrier()

        @pl.when(core == 0)
        def _():
            edge_sweep(fsrc_list[c], src1d, dstl3d)

        @pl.when(core == 1)
        def _():
            edge_sweep(fdst_list[c], dstl1d, src3d)

        plsc.subcore_barrier()

        rowbase = jnp.where(core == 0, N_USERS, 0)

        @pl.when(s < NS - 1)
        def _():
            pltpu.sync_copy(acc_sh.at[pl.ds(s * TROWS, TROWS)],
                            out4_hbm.at[c, pl.ds(rowbase + s * TROWS, TROWS)])

        @pl.when(s == NS - 1)
        def _():
            last = N_USERS - (NS - 1) * TROWS
            pltpu.sync_copy(acc_sh.at[pl.ds((NS - 1) * TROWS, last)],
                            out4_hbm.at[c, pl.ds(rowbase + (NS - 1) * TROWS, last)])

        plsc.subcore_barrier()


def _aggregate(src1d, dstl1d, p1d, src3d, dstl3d, fsrc_list, fdst_list):
    mesh = plsc.VectorSubcoreMesh(core_axis_name="c", subcore_axis_name="s")

    def body(src1d_r, dstl1d_r, p1d_r, src3d_r, dstl3d_r,
             f0, f1, f2, f3, g0, g1, g2, g3, out4_r,
             gidx_v, sidx_v, grows_v, p_v, zbuf_v, acc_sh, semi, semg, semsc):
        _accum_body(src1d_r, dstl1d_r, p1d_r, src3d_r, dstl3d_r,
                    [f0, f1, f2, f3], [g0, g1, g2, g3], out4_r,
                    gidx_v, sidx_v, grows_v, p_v, zbuf_v, acc_sh,
                    semi, semg, semsc)

    return pl.kernel(
        body,
        out_type=jax.ShapeDtypeStruct((NCHUNK, 2 * N_USERS, CHUNK), jnp.float32),
        mesh=mesh,
        compiler_params=pltpu.CompilerParams(use_tc_tiling_on_sc=False),
        scratch_types=[
            pltpu.VMEM((2, 1, B2), jnp.int32),
            pltpu.VMEM((3 * (B2 // 128), 1, 128), jnp.int32),
            pltpu.VMEM((2, B2, CHUNK), jnp.float32),
            pltpu.VMEM((2, 1, B2), jnp.float32),
            pltpu.VMEM((ZROWS, CHUNK), jnp.float32),
            pltpu.VMEM_SHARED((ACC_ROWS, CHUNK), jnp.float32),
            pltpu.SemaphoreType.DMA,
            pltpu.SemaphoreType.DMA,
            pltpu.SemaphoreType.DMA,
        ],
    )(src1d, dstl1d, p1d, src3d, dstl3d, *fsrc_list, *fdst_list)


def kernel(feat, edge_index, user_ids, item_ids, W_src, b_src, W_dst, b_dst):
    feat = feat.astype(jnp.float32)
    src = edge_index[0].astype(jnp.int32)
    dstl = edge_index[1].astype(jnp.int32)
    dstg = dstl + N_USERS

    t_all = _transform(feat, W_src.T, b_src.reshape(1, D), W_dst.T,
                       b_dst.reshape(1, D))

    lpart = _edge_logits(src.reshape(E // B1, 1, B1),
                         dstg.reshape(E // B1, 1, B1), feat)
    sel = jnp.repeat(jnp.eye(_RCOLS // 16, dtype=jnp.float32), 16, axis=0)
    logits2d = _rowsum(lpart.reshape(E * 16 // _RCOLS, _RCOLS), sel)
    p = _softmax(logits2d).reshape(-1)

    fsrc_list = [t_all[:N_USERS, c * CHUNK:(c + 1) * CHUNK] for c in range(NCHUNK)]
    fdst_list = [t_all[N_USERS:, c * CHUNK:(c + 1) * CHUNK] for c in range(NCHUNK)]
    src_g3 = src.reshape(NBLK, 1, B2)
    dstl_g3 = dstl.reshape(NBLK, 1, B2)
    p3 = p.reshape(NBLK, 1, B2)
    src_s3 = src.reshape(NBLK * (B2 // 128), 1, 128)
    dstl_s3 = dstl.reshape(NBLK * (B2 // 128), 1, 128)
    out4 = _aggregate(src_g3, dstl_g3, p3, src_s3, dstl_s3,
                      fsrc_list, fdst_list)
    return out4.transpose(1, 0, 2).reshape(2 * N_USERS, D)


# speedup vs baseline: 1.8781x; 1.1349x over previous
import math

import jax
import jax.numpy as jnp
from jax import lax
from jax.experimental import pallas as pl
from jax.experimental.pallas import tpu as pltpu
from jax.experimental.pallas import tpu_sc as plsc

N_USERS = 50000
N_ITEMS = 50000
D = 128
E = 1600000

NC, NS = 2, 16
NW = NC * NS
EW = E // NW
B1 = 200
SUB = 40
NSUB = B1 // SUB

CHUNK = 32
NCHUNK = D // CHUNK
B2 = 320
SUB2 = 80
NBLK = E // B2
ACC_ROWS = 51200
TROWS = ACC_ROWS // NS
ZROWS = 64

_INV_SQRT_D = 1.0 / math.sqrt(float(D))


_TBLK = 400


def _transform_body(x_ref, wu_ref, bu_ref, wd_ref, bd_ref, o_ref):
    i = pl.program_id(0)
    x = x_ref[...]
    yu = jnp.maximum(jnp.dot(x, wu_ref[...], preferred_element_type=jnp.float32)
                     + bu_ref[...], 0.0)
    yd = jnp.maximum(jnp.dot(x, wd_ref[...], preferred_element_type=jnp.float32)
                     + bd_ref[...], 0.0)
    o_ref[...] = jnp.where(i < (N_USERS // _TBLK), yu, yd)


def _transform(feat, wu_t, bu, wd_t, bd):
    n = feat.shape[0]
    return pl.pallas_call(
        _transform_body,
        grid=(n // _TBLK,),
        in_specs=[
            pl.BlockSpec((_TBLK, D), lambda i: (i, 0)),
            pl.BlockSpec((D, D), lambda i: (0, 0)),
            pl.BlockSpec((1, D), lambda i: (0, 0)),
            pl.BlockSpec((D, D), lambda i: (0, 0)),
            pl.BlockSpec((1, D), lambda i: (0, 0)),
        ],
        out_specs=pl.BlockSpec((_TBLK, D), lambda i: (i, 0)),
        out_shape=jax.ShapeDtypeStruct((n, D), jnp.float32),
    )(feat, wu_t, bu, wd_t, bd)


def _softmax_body(l_ref, p_ref):
    l = l_ref[...] * _INV_SQRT_D
    m = jnp.max(l)
    ex = jnp.exp(l - m)
    z = jnp.sum(ex)
    p_ref[...] = ex / z


def _softmax(logits2d):
    return pl.pallas_call(
        _softmax_body,
        out_shape=jax.ShapeDtypeStruct(logits2d.shape, jnp.float32),
    )(logits2d)


def _logits_body(srcg3, dstg3, feat_hbm, out_hbm,
                 idxu_v, idxv_v, urows_v, vrows_v, lbuf_v, semi, semg, semo):
    wid = lax.axis_index("s") * NC + lax.axis_index("c")
    row0 = wid * (EW // B1)
    nr = EW // B1

    def issue_idx(r, slot):
        pltpu.async_copy(srcg3.at[pl.ds(row0 + r, 1)],
                         idxu_v.at[pl.ds(slot, 1)], semi)
        pltpu.async_copy(dstg3.at[pl.ds(row0 + r, 1)],
                         idxv_v.at[pl.ds(slot, 1)], semi)

    def wait_idx():
        pltpu.make_async_copy(srcg3.at[pl.ds(row0, 1)],
                              idxu_v.at[pl.ds(0, 1)], semi).wait()
        pltpu.make_async_copy(dstg3.at[pl.ds(row0, 1)],
                              idxv_v.at[pl.ds(0, 1)], semi).wait()

    def issue_gathers(slot):
        for j in range(NSUB):
            pltpu.async_copy(
                feat_hbm.at[idxu_v.at[slot, 0, pl.ds(j * SUB, SUB)]],
                urows_v.at[slot, pl.ds(j * SUB, SUB)], semg)
            pltpu.async_copy(
                feat_hbm.at[idxv_v.at[slot, 0, pl.ds(j * SUB, SUB)]],
                vrows_v.at[slot, pl.ds(j * SUB, SUB)], semg)

    def wait_gathers():
        for j in range(2 * NSUB):
            pltpu.make_async_copy(
                feat_hbm.at[idxu_v.at[0, 0, pl.ds(0, SUB)]],
                urows_v.at[0, pl.ds(0, SUB)], semg).wait()

    def wait_out():
        pltpu.make_async_copy(lbuf_v.at[0], out_hbm.at[pl.ds(0, B1)],
                              semo).wait()

    issue_idx(0, 0)
    issue_idx(1, 1)
    wait_idx()
    issue_gathers(0)

    def round_body(r, _):
        slot = lax.rem(r, 2)
        wait_gathers()

        @pl.when(r + 1 < nr)
        def _():
            wait_idx()
            issue_gathers(1 - slot)

        def edge_body(e, _):
            acc = (urows_v[slot, e, pl.ds(0, 16)]
                   * vrows_v[slot, e, pl.ds(0, 16)])
            for si in range(1, 8):
                acc = acc + (urows_v[slot, e, pl.ds(16 * si, 16)]
                             * vrows_v[slot, e, pl.ds(16 * si, 16)])
            lbuf_v[slot, e, pl.ds(0, 16)] = acc
            return 0

        lax.fori_loop(0, B1, edge_body, 0, unroll=False)
        pltpu.async_copy(lbuf_v.at[slot],
                         out_hbm.at[pl.ds((row0 + r) * B1, B1)], semo)

        @pl.when(r >= 1)
        def _():
            wait_out()

        @pl.when(r + 2 < nr)
        def _():
            issue_idx(r + 2, slot)

        return 0

    lax.fori_loop(0, nr, round_body, 0, unroll=False)
    wait_out()


def _edge_logits(srcg3, dstg3, feat):
    mesh = plsc.VectorSubcoreMesh(core_axis_name="c", subcore_axis_name="s")
    return pl.kernel(
        _logits_body,
        out_type=jax.ShapeDtypeStruct((E, 16), jnp.float32),
        mesh=mesh,
        compiler_params=pltpu.CompilerParams(use_tc_tiling_on_sc=False),
        scratch_types=[
            pltpu.VMEM((2, 1, B1), jnp.int32),
            pltpu.VMEM((2, 1, B1), jnp.int32),
            pltpu.VMEM((2, B1, D), jnp.float32),
            pltpu.VMEM((2, B1, D), jnp.float32),
            pltpu.VMEM((2, B1, 16), jnp.float32),
            pltpu.SemaphoreType.DMA,
            pltpu.SemaphoreType.DMA,
            pltpu.SemaphoreType.DMA,
        ],
    )(srcg3, dstg3, feat)


_RCOLS = 1024
_RBLK = 200


def _rowsum_body(x_ref, s_ref, o_ref):
    o_ref[...] = jnp.dot(x_ref[...], s_ref[...],
                         preferred_element_type=jnp.float32)


def _rowsum(part2d, sel):
    n = part2d.shape[0]
    return pl.pallas_call(
        _rowsum_body,
        grid=(n // _RBLK,),
        in_specs=[
            pl.BlockSpec((_RBLK, _RCOLS), lambda i: (i, 0)),
            pl.BlockSpec((_RCOLS, _RCOLS // 16), lambda i: (0, 0)),
        ],
        out_specs=pl.BlockSpec((_RBLK, _RCOLS // 16), lambda i: (i, 0)),
        out_shape=jax.ShapeDtypeStruct((n, _RCOLS // 16), jnp.float32),
    )(part2d, sel)


def _accum_body(src1d, dstl1d, p1d, src3d, dstl3d,
                fsrc_list, fdst_list, out4_hbm,
                gidx_v, sidx_v, grows_v, p_v, zbuf_v, acc_sh,
                semi, semg, semsc):
    core = lax.axis_index("c")
    s = lax.axis_index("s")
    lane = lax.iota(jnp.int32, 16)
    zeros16 = jnp.zeros((16,), jnp.float32)

    def zrow(i, _):
        zbuf_v[i, pl.ds(0, 16)] = zeros16
        zbuf_v[i, pl.ds(16, 16)] = zeros16
        return 0

    lax.fori_loop(0, ZROWS, zrow, 0, unroll=False)

    nb = (NBLK - s + NS - 1) // NS
    _NJ = B2 // SUB2

    def edge_sweep(gather_ref, gidx_hbm, sidx_hbm):
        def issue_idx(t, gslot, sslot):
            b = s + t * NS
            pltpu.async_copy(gidx_hbm.at[pl.ds(b, 1)],
                             gidx_v.at[pl.ds(gslot, 1)], semi)
            pltpu.async_copy(p1d.at[pl.ds(b, 1)], p_v.at[pl.ds(gslot, 1)], semi)
            pltpu.async_copy(sidx_hbm.at[pl.ds(b * _NJ, _NJ)],
                             sidx_v.at[pl.ds(sslot * _NJ, _NJ)], semi)

        def wait_idx():
            pltpu.make_async_copy(gidx_hbm.at[pl.ds(s, 1)],
                                  gidx_v.at[pl.ds(0, 1)], semi).wait()
            pltpu.make_async_copy(p1d.at[pl.ds(s, 1)],
                                  p_v.at[pl.ds(0, 1)], semi).wait()
            pltpu.make_async_copy(sidx_hbm.at[pl.ds(s, _NJ)],
                                  sidx_v.at[pl.ds(0, _NJ)], semi).wait()

        def issue_gather(slot):
            for j in range(_NJ):
                pltpu.async_copy(
                    gather_ref.at[gidx_v.at[slot, 0, pl.ds(j * SUB2, SUB2)]],
                    grows_v.at[slot, pl.ds(j * SUB2, SUB2)], semg)

        def wait_gather():
            for j in range(_NJ):
                pltpu.make_async_copy(
                    gather_ref.at[gidx_v.at[0, 0, pl.ds(j * SUB2, SUB2)]],
                    grows_v.at[0, pl.ds(j * SUB2, SUB2)], semg).wait()

        def issue_scatter(slot, s3):
            for j in range(_NJ):
                pltpu.async_copy(grows_v.at[slot, pl.ds(j * SUB2, SUB2)],
                                 acc_sh.at[sidx_v.at[s3 * _NJ + j, 0]], semsc,
                                 add=True)

        def wait_scatter():
            for j in range(_NJ):
                pltpu.make_async_copy(grows_v.at[0, pl.ds(j * SUB2, SUB2)],
                                      acc_sh.at[sidx_v.at[j, 0]], semsc).wait()

        def scale(slot):
            def scale_group(g, _):
                pv16 = p_v[slot, 0, pl.ds(g * 16, 16)]
                base = g * 16
                for k in range(16):
                    pv = pv16[k]
                    for si in range(CHUNK // 16):
                        sl = pl.ds(16 * si, 16)
                        grows_v[slot, base + k, sl] = (
                            grows_v[slot, base + k, sl] * pv)
                return 0

            lax.fori_loop(0, B2 // 16, scale_group, 0, unroll=False)

        issue_idx(0, 0, 0)
        issue_idx(1, 1, 1)
        wait_idx()
        issue_gather(0)

        def blk(t, _):
            slot = lax.rem(t, 2)
            s3 = lax.rem(t, 3)
            wait_gather()

            @pl.when(t >= 1)
            def _():
                wait_scatter()

            @pl.when(t + 1 < nb)
            def _():
                wait_idx()
                issue_gather(1 - slot)

            scale(slot)
            issue_scatter(slot, s3)

            @pl.when(t + 2 < nb)
            def _():
                issue_idx(t + 2, slot, lax.rem(t + 2, 3))

            return 0

        lax.fori_loop(0, nb, blk, 0, unroll=False)
        wait_scatter()

    for c in range(NCHUNK):
        for k in range(TROWS // ZROWS):
            pltpu.sync_copy(zbuf_v, acc_sh.at[pl.ds(s * TROWS + k * ZROWS, ZROWS)])
        plsc.subcore_barrier()

        @pl.when(core == 0)
        def _():
            edge_sweep(fsrc_list[c], src1d, dstl3d)

        @pl.when(core == 1)
        def _():
            edge_sweep(fdst_list[c], dstl1d, src3d)

        plsc.subcore_barrier()

        rowbase = jnp.where(core == 0, N_USERS, 0)

        @pl.when(s < NS - 1)
        def _():
            pltpu.sync_copy(acc_sh.at[pl.ds(s * TROWS, TROWS)],
                            out4_hbm.at[pl.ds(rowbase + s * TROWS, TROWS),
                                        pl.ds(c * CHUNK, CHUNK)])

        @pl.when(s == NS - 1)
        def _():
            last = N_USERS - (NS - 1) * TROWS
            pltpu.sync_copy(acc_sh.at[pl.ds((NS - 1) * TROWS, last)],
                            out4_hbm.at[pl.ds(rowbase + (NS - 1) * TROWS, last),
                                        pl.ds(c * CHUNK, CHUNK)])

        plsc.subcore_barrier()


def _aggregate(src1d, dstl1d, p1d, src3d, dstl3d, fsrc_list, fdst_list):
    mesh = plsc.VectorSubcoreMesh(core_axis_name="c", subcore_axis_name="s")

    def body(src1d_r, dstl1d_r, p1d_r, src3d_r, dstl3d_r,
             f0, f1, f2, f3, g0, g1, g2, g3, out4_r,
             gidx_v, sidx_v, grows_v, p_v, zbuf_v, acc_sh, semi, semg, semsc):
        _accum_body(src1d_r, dstl1d_r, p1d_r, src3d_r, dstl3d_r,
                    [f0, f1, f2, f3], [g0, g1, g2, g3], out4_r,
                    gidx_v, sidx_v, grows_v, p_v, zbuf_v, acc_sh,
                    semi, semg, semsc)

    return pl.kernel(
        body,
        out_type=jax.ShapeDtypeStruct((2 * N_USERS, D), jnp.float32),
        mesh=mesh,
        compiler_params=pltpu.CompilerParams(use_tc_tiling_on_sc=False),
        scratch_types=[
            pltpu.VMEM((2, 1, B2), jnp.int32),
            pltpu.VMEM((3 * (B2 // SUB2), 1, SUB2), jnp.int32),
            pltpu.VMEM((2, B2, CHUNK), jnp.float32),
            pltpu.VMEM((2, 1, B2), jnp.float32),
            pltpu.VMEM((ZROWS, CHUNK), jnp.float32),
            pltpu.VMEM_SHARED((ACC_ROWS, CHUNK), jnp.float32),
            pltpu.SemaphoreType.DMA,
            pltpu.SemaphoreType.DMA,
            pltpu.SemaphoreType.DMA,
        ],
    )(src1d, dstl1d, p1d, src3d, dstl3d, *fsrc_list, *fdst_list)


def kernel(feat, edge_index, user_ids, item_ids, W_src, b_src, W_dst, b_dst):
    feat = feat.astype(jnp.float32)
    src = edge_index[0].astype(jnp.int32)
    dstl = edge_index[1].astype(jnp.int32)
    dstg = dstl + N_USERS

    t_all = _transform(feat, W_src.T, b_src.reshape(1, D), W_dst.T,
                       b_dst.reshape(1, D))

    lpart = _edge_logits(src.reshape(E // B1, 1, B1),
                         dstg.reshape(E // B1, 1, B1), feat)
    sel = jnp.repeat(jnp.eye(_RCOLS // 16, dtype=jnp.float32), 16, axis=0)
    logits2d = _rowsum(lpart.reshape(E * 16 // _RCOLS, _RCOLS), sel)
    p = _softmax(logits2d).reshape(-1)

    fsrc_list = [t_all[:N_USERS, c * CHUNK:(c + 1) * CHUNK] for c in range(NCHUNK)]
    fdst_list = [t_all[N_USERS:, c * CHUNK:(c + 1) * CHUNK] for c in range(NCHUNK)]
    src_g3 = src.reshape(NBLK, 1, B2)
    dstl_g3 = dstl.reshape(NBLK, 1, B2)
    p3 = p.reshape(NBLK, 1, B2)
    src_s3 = src.reshape(NBLK * (B2 // SUB2), 1, SUB2)
    dstl_s3 = dstl.reshape(NBLK * (B2 // SUB2), 1, SUB2)
    return _aggregate(src_g3, dstl_g3, p3, src_s3, dstl_s3,
                      fsrc_list, fdst_list)
